# Initial kernel scaffold; baseline (speedup 1.0000x reference)
import jax
import jax.numpy as jnp
from jax.experimental import pallas as pl


def kernel(x, edge_index, edge_attr, batch, W0, b0, Wh, bh, bn_gamma, bn_beta, bn_mean, bn_var, lin_w, lin_b):
    raise NotImplementedError("write your pallas kernel here")


# speedup vs baseline: 1.1336x; 1.1336x over previous
import functools

import jax
import jax.numpy as jnp
from jax.experimental import pallas as pl

_N, _E, _D, _H, _L, _G = 10000, 160000, 256, 512, 4, 64


def _mm_kernel(agg_ref, ea_ref, wt_ref, wb_ref, s_ref, t_ref, out_ref):
    acc = jnp.dot(agg_ref[...], wt_ref[...], preferred_element_type=jnp.float32,
                  precision=jax.lax.Precision.HIGHEST)
    acc += jnp.dot(ea_ref[...], wb_ref[...], preferred_element_type=jnp.float32,
                   precision=jax.lax.Precision.HIGHEST)
    out_ref[...] = jnp.maximum(acc * s_ref[...] + t_ref[...], 0.0)


def _layer_update(agg, ea, wt, wb, s, t):
    n, hin = agg.shape
    h = wt.shape[1]
    bn = 400
    return pl.pallas_call(
        _mm_kernel,
        grid=(n // bn,),
        in_specs=[
            pl.BlockSpec((bn, hin), lambda i: (i, 0)),
            pl.BlockSpec((bn, 8), lambda i: (i, 0)),
            pl.BlockSpec((hin, h), lambda i: (0, 0)),
            pl.BlockSpec((8, h), lambda i: (0, 0)),
            pl.BlockSpec((1, h), lambda i: (0, 0)),
            pl.BlockSpec((1, h), lambda i: (0, 0)),
        ],
        out_specs=pl.BlockSpec((bn, h), lambda i: (i, 0)),
        out_shape=jax.ShapeDtypeStruct((n, h), jnp.float32),
    )(agg, ea, wt, wb, s.reshape(1, -1), t.reshape(1, -1))


def kernel(x, edge_index, edge_attr, batch, W0, b0, Wh, bh, bn_gamma, bn_beta,
           bn_mean, bn_var, lin_w, lin_b):
    src = edge_index[0]
    dst = edge_index[1]

    ea = jax.ops.segment_max(edge_attr, dst, num_segments=_N)
    ea = jnp.where(jnp.isneginf(ea), 0.0, ea)
    ea = jnp.pad(ea, ((0, 0), (0, 2)))

    inv_sd = bn_gamma / jnp.sqrt(bn_var + 1e-5)

    h = x
    for i in range(_L):
        agg = jax.ops.segment_max(h[src], dst, num_segments=_N)
        agg = jnp.where(jnp.isneginf(agg), 0.0, agg)
        w = W0 if i == 0 else Wh[i - 1]
        b = b0 if i == 0 else bh[i - 1]
        hin = w.shape[0] - 6
        wt = w[:hin]
        wb = jnp.pad(w[hin:], ((0, 2), (0, 0)))
        s = inv_sd[i]
        t = (b - bn_mean[i]) * s + bn_beta[i]
        h = _layer_update(agg, ea, wt, wb, s, t)

    pooled = jax.ops.segment_max(h, batch, num_segments=_G)
    pooled = jnp.where(jnp.isneginf(pooled), 0.0, pooled)
    return pooled @ lin_w + lin_b


# speedup vs baseline: 1.8454x; 1.6279x over previous
import functools

import jax
import jax.numpy as jnp
from jax import lax
from jax.experimental import pallas as pl
from jax.experimental.pallas import tpu as pltpu
from jax.experimental.pallas import tpu_sc as plsc

_N, _E, _D, _H, _L, _G = 10000, 160000, 256, 512, 4, 64

_NC, _NS = 2, 16
_NW = _NC * _NS
_NB = 64
_RB = 160
_NPAD = _NB * _RB
_C = 64

_mesh = plsc.VectorSubcoreMesh(core_axis_name="c", subcore_axis_name="s")


def _wid():
    return lax.axis_index("s") * _NC + lax.axis_index("c")


def _make_agg(hin, indirect=True):
    def body(table_hbm, bpk_hbm, starts_hbm, out_hbm,
             accv, rowsv, pkv, idxv, st_smem, sem):
        w = _wid()
        nj = hin // 16

        pltpu.sync_copy(starts_hbm, st_smem.at[pl.ds(0, 80)])

        def bucket_body(k, _):
            b = 2 * w + k
            start = st_smem[pl.ds(b, 16)][0]
            end = st_smem[pl.ds(b + 1, 16)][0]

            def init(r, _):
                for j in range(nj):
                    accv[r, pl.ds(j * 16, 16)] = jnp.full((16,), -jnp.inf,
                                                          jnp.float32)
                return 0

            lax.fori_loop(0, _RB, init, 0)

            a0 = (start // _C) * _C
            nch = (end - a0 + (_C - 1)) // _C

            def chunk_body(c, _):
                cbase = a0 + c * _C
                pk_off = pl.multiple_of(cbase, _C)
                pltpu.sync_copy(bpk_hbm.at[pl.ds(pk_off, _C)],
                                pkv.at[pl.ds(0, _C)])
                if indirect:
                    for i in range(_C // 16):
                        v = pkv[pl.ds(i * 16, 16)]
                        idxv[pl.ds(i * 16, 16)] = v >> 8
                    pltpu.async_copy(table_hbm.at[idxv], rowsv, sem).wait()
                else:
                    pltpu.sync_copy(table_hbm.at[pl.ds(pk_off, _C)], rowsv)
                elo = jnp.maximum(0, start - cbase)
                ehi = jnp.minimum(_C, end - cbase)

                def edge_body(e, _):
                    dl = pkv[pl.ds(e, 16)][0] & 255
                    for j in range(nj):
                        a = accv[dl, pl.ds(j * 16, 16)]
                        mrow = rowsv[e, pl.ds(j * 16, 16)]
                        accv[dl, pl.ds(j * 16, 16)] = jnp.maximum(a, mrow)
                    return 0

                lax.fori_loop(elo, ehi, edge_body, 0)
                return 0

            lax.fori_loop(0, nch, chunk_body, 0)

            def fix(r, _):
                for j in range(nj):
                    v = accv[r, pl.ds(j * 16, 16)]
                    accv[r, pl.ds(j * 16, 16)] = jnp.where(
                        v == -jnp.inf, 0.0, v)
                return 0

            lax.fori_loop(0, _RB, fix, 0)
            pltpu.sync_copy(accv, out_hbm.at[pl.ds(b * _RB, _RB)])
            return 0

        lax.fori_loop(0, 2, bucket_body, 0)

    return pl.kernel(
        body,
        out_type=jax.ShapeDtypeStruct((_NPAD, hin), jnp.float32),
        mesh=_mesh,
        scratch_types=[pltpu.VMEM((_RB, hin), jnp.float32),
                       pltpu.VMEM((_C, hin), jnp.float32),
                       pltpu.VMEM((_C + 16,), jnp.int32),
                       pltpu.VMEM((_C,), jnp.int32),
                       pltpu.VMEM((96,), jnp.int32),
                       pltpu.SemaphoreType.DMA],
    )


def _pool_body(h_hbm, batch_hbm, part_hbm, accv, rowsv, bt_smem, sem):
    w = _wid()
    r0 = pl.multiple_of(w * 320, 8)
    nr = jnp.where(w == _NW - 1, _N - 320 * (_NW - 1), 320)

    def init(r, _):
        for j in range(_H // 16):
            accv[r, pl.ds(j * 16, 16)] = jnp.full((16,), -jnp.inf, jnp.float32)
        return 0

    lax.fori_loop(0, _G, init, 0)

    @pl.when(w < _NW - 1)
    def _():
        pltpu.sync_copy(batch_hbm.at[pl.ds(r0, 320)],
                        bt_smem.at[pl.ds(0, 320)])

    @pl.when(w == _NW - 1)
    def _():
        pltpu.sync_copy(batch_hbm.at[pl.ds(320 * (_NW - 1),
                                           _N - 320 * (_NW - 1))],
                        bt_smem.at[pl.ds(0, _N - 320 * (_NW - 1))])

    def chunk_body(c, _):
        pltpu.sync_copy(h_hbm.at[pl.ds(r0 + c * 16, 16)], rowsv)

        def row_body(e, _):
            g = bt_smem[pl.ds(c * 16 + e, 16)][0]
            for j in range(_H // 16):
                a = accv[g, pl.ds(j * 16, 16)]
                m = rowsv[e, pl.ds(j * 16, 16)]
                accv[g, pl.ds(j * 16, 16)] = jnp.maximum(a, m)
            return 0

        lax.fori_loop(0, 16, row_body, 0)
        return 0

    lax.fori_loop(0, nr // 16, chunk_body, 0)
    pltpu.sync_copy(accv, part_hbm.at[w])


_pool = pl.kernel(
    _pool_body,
    out_type=jax.ShapeDtypeStruct((_NW, _G, _H), jnp.float32),
    mesh=_mesh,
    scratch_types=[pltpu.VMEM((_G, _H), jnp.float32),
                   pltpu.VMEM((16, _H), jnp.float32),
                   pltpu.VMEM((336,), jnp.int32),
                   pltpu.SemaphoreType.DMA],
)


def _mm_kernel(agg_ref, ea_ref, wt_ref, wb_ref, s_ref, t_ref, out_ref):
    acc = jnp.dot(agg_ref[...], wt_ref[...], preferred_element_type=jnp.float32)
    acc += jnp.dot(ea_ref[...], wb_ref[...], preferred_element_type=jnp.float32)
    out_ref[...] = jnp.maximum(acc * s_ref[...] + t_ref[...], 0.0)


def _layer_update(agg, ea, wt, wb, s, t):
    hin = wt.shape[0]
    h = wt.shape[1]
    bn = 400
    return pl.pallas_call(
        _mm_kernel,
        grid=(_N // bn,),
        in_specs=[
            pl.BlockSpec((bn, hin), lambda i: (i, 0)),
            pl.BlockSpec((bn, 16), lambda i: (i, 0)),
            pl.BlockSpec((hin, h), lambda i: (0, 0)),
            pl.BlockSpec((16, h), lambda i: (0, 0)),
            pl.BlockSpec((1, h), lambda i: (0, 0)),
            pl.BlockSpec((1, h), lambda i: (0, 0)),
        ],
        out_specs=pl.BlockSpec((bn, h), lambda i: (i, 0)),
        out_shape=jax.ShapeDtypeStruct((_N, h), jnp.float32),
    )(agg, ea, wt, wb, s.reshape(1, -1), t.reshape(1, -1))


def _final_kernel(part_ref, w_ref, b_ref, out_ref):
    p = jnp.max(part_ref[...], axis=0)
    p = jnp.where(jnp.isneginf(p), 0.0, p)
    out_ref[...] = jnp.dot(p, w_ref[...],
                           preferred_element_type=jnp.float32) + b_ref[...]


def _final(part, lin_w_pad, lin_b):
    return pl.pallas_call(
        _final_kernel,
        out_shape=jax.ShapeDtypeStruct((_G, 128), jnp.float32),
    )(part, lin_w_pad, lin_b.reshape(1, 1) * jnp.ones((1, 128), jnp.float32))


def kernel(x, edge_index, edge_attr, batch, W0, b0, Wh, bh, bn_gamma, bn_beta,
           bn_mean, bn_var, lin_w, lin_b):
    src = edge_index[0]
    dst = edge_index[1]

    perm = jnp.argsort(dst).astype(jnp.int32)
    sdst = dst[perm]
    ssrc = src[perm]
    dl = sdst % _RB
    pad = jnp.zeros((_C,), jnp.int32)
    pk_src = jnp.concatenate([ssrc * 256 + dl, pad])
    pk_dl = jnp.concatenate([dl, pad])
    edges = jnp.arange(0, _NB * _RB + 1, _RB, dtype=jnp.int32)
    starts = jnp.zeros((80,), jnp.int32).at[:_NB + 1].set(
        jnp.searchsorted(sdst, edges).astype(jnp.int32))

    ea_sorted = jnp.pad(edge_attr[perm], ((0, _C), (0, 10)))
    ea_agg = _make_agg(16, indirect=False)(ea_sorted, pk_dl, starts)

    inv_sd = bn_gamma / jnp.sqrt(bn_var + 1e-5)

    h = x
    for i in range(_L):
        hin = _D if i == 0 else _H
        agg = _make_agg(hin)(h, pk_src, starts)
        w = W0 if i == 0 else Wh[i - 1]
        b = b0 if i == 0 else bh[i - 1]
        wt = w[:hin]
        wb = jnp.pad(w[hin:], ((0, 10), (0, 0)))
        s = inv_sd[i]
        t = (b - bn_mean[i]) * s + bn_beta[i]
        h = _layer_update(agg, ea_agg, wt, wb, s, t)

    part = _pool(h, batch)
    out = _final(part, jnp.pad(lin_w, ((0, 0), (0, 127))), lin_b)
    return out[:, :1]


# speedup vs baseline: 2.2421x; 1.2150x over previous
import functools

import jax
import jax.numpy as jnp
from jax import lax
from jax.experimental import pallas as pl
from jax.experimental.pallas import tpu as pltpu
from jax.experimental.pallas import tpu_sc as plsc

_N, _E, _D, _H, _L, _G = 10000, 160000, 256, 512, 4, 64

_NC, _NS = 2, 16
_NW = _NC * _NS
_NB = 64
_RB = 160
_NPAD = _NB * _RB
_C = 64
_SUP = 4096

_mesh = plsc.VectorSubcoreMesh(core_axis_name="c", subcore_axis_name="s")


def _wid():
    return lax.axis_index("s") * _NC + lax.axis_index("c")


def _make_agg(hin, indirect=True):
    cc = _C if hin <= 256 else 32

    def body(table_hbm, bpk_hbm, starts_hbm, out_hbm,
             accv, rows0, rows1, pkv, idx0, idx1, st_smem, sem0, sem1):
        w = _wid()
        nj = hin // 16

        pltpu.sync_copy(starts_hbm, st_smem.at[pl.ds(0, 80)])

        def bucket_body(k, _):
            b = 2 * w + k
            start = st_smem[pl.ds(b, 16)][0]
            end = st_smem[pl.ds(b + 1, 16)][0]

            def init(r, _):
                for j in range(nj):
                    accv[r, pl.ds(j * 16, 16)] = jnp.full((16,), -jnp.inf,
                                                          jnp.float32)
                return 0

            lax.fori_loop(0, _RB, init, 0)

            a0 = (start // cc) * cc
            nsup = (end - a0 + (_SUP - 1)) // _SUP

            def issue(sbase, c, idxv, rbuf, sem):
                if indirect:
                    for i in range(cc // 16):
                        v = pkv[pl.ds(c * cc + i * 16, 16)]
                        idxv[pl.ds(i * 16, 16)] = v >> 8
                    pltpu.async_copy(table_hbm.at[idxv], rbuf, sem)
                else:
                    g_off = pl.multiple_of(sbase + c * cc, cc)
                    pltpu.async_copy(table_hbm.at[pl.ds(g_off, cc)],
                                     rbuf, sem)

            def wait(idxv, rbuf, sem):
                if indirect:
                    pltpu.make_async_copy(table_hbm.at[idxv], rbuf,
                                          sem).wait()
                else:
                    pltpu.make_async_copy(table_hbm.at[pl.ds(0, cc)],
                                          rbuf, sem).wait()

            def process(sbase, c, rbuf):
                cbase = sbase + c * cc
                elo = jnp.maximum(0, start - cbase)
                ehi = jnp.minimum(cc, end - cbase)

                def edge_body(e, _):
                    dl = pkv[pl.ds(c * cc + e, 16)][0] & 255
                    for j in range(nj):
                        a = accv[dl, pl.ds(j * 16, 16)]
                        mrow = rbuf[e, pl.ds(j * 16, 16)]
                        accv[dl, pl.ds(j * 16, 16)] = jnp.maximum(a, mrow)
                    return 0

                lax.fori_loop(elo, ehi, edge_body, 0)

            def sup_body(s_, _):
                sbase = a0 + s_ * _SUP
                sup_off = pl.multiple_of(sbase, cc)
                pltpu.sync_copy(bpk_hbm.at[pl.ds(sup_off, _SUP)],
                                pkv.at[pl.ds(0, _SUP)])
                nch = (jnp.minimum(_SUP, end - sbase) + (cc - 1)) // cc

                @pl.when(nch > 0)
                def _():
                    issue(sbase, 0, idx0, rows0, sem0)

                    def pair_body(p, _):
                        c0 = 2 * p
                        c1 = c0 + 1

                        @pl.when(c1 < nch)
                        def _():
                            issue(sbase, c1, idx1, rows1, sem1)

                        wait(idx0, rows0, sem0)
                        process(sbase, c0, rows0)

                        @pl.when(c1 < nch)
                        def _():
                            @pl.when(c1 + 1 < nch)
                            def _():
                                issue(sbase, c1 + 1, idx0, rows0, sem0)

                            wait(idx1, rows1, sem1)
                            process(sbase, c1, rows1)

                        return 0

                    lax.fori_loop(0, (nch + 1) // 2, pair_body, 0)

                return 0

            lax.fori_loop(0, nsup, sup_body, 0)

            def fix(r, _):
                for j in range(nj):
                    v = accv[r, pl.ds(j * 16, 16)]
                    accv[r, pl.ds(j * 16, 16)] = jnp.where(
                        v == -jnp.inf, 0.0, v)
                return 0

            lax.fori_loop(0, _RB, fix, 0)
            pltpu.sync_copy(accv, out_hbm.at[pl.ds(b * _RB, _RB)])
            return 0

        lax.fori_loop(0, 2, bucket_body, 0)

    return pl.kernel(
        body,
        out_type=jax.ShapeDtypeStruct((_NPAD, hin), jnp.float32),
        mesh=_mesh,
        scratch_types=[pltpu.VMEM((_RB, hin), jnp.float32),
                       pltpu.VMEM((cc, hin), jnp.float32),
                       pltpu.VMEM((cc, hin), jnp.float32),
                       pltpu.VMEM((_SUP + 16,), jnp.int32),
                       pltpu.VMEM((cc,), jnp.int32),
                       pltpu.VMEM((cc,), jnp.int32),
                       pltpu.VMEM((96,), jnp.int32),
                       pltpu.SemaphoreType.DMA,
                       pltpu.SemaphoreType.DMA],
    )


def _pool_body(h_hbm, batch_hbm, part_hbm, accv, rowsv, bt_smem, sem):
    w = _wid()
    r0 = pl.multiple_of(w * 320, 8)
    nr = jnp.where(w == _NW - 1, _N - 320 * (_NW - 1), 320)

    def init(r, _):
        for j in range(_H // 16):
            accv[r, pl.ds(j * 16, 16)] = jnp.full((16,), -jnp.inf, jnp.float32)
        return 0

    lax.fori_loop(0, _G, init, 0)

    @pl.when(w < _NW - 1)
    def _():
        pltpu.sync_copy(batch_hbm.at[pl.ds(r0, 320)],
                        bt_smem.at[pl.ds(0, 320)])

    @pl.when(w == _NW - 1)
    def _():
        pltpu.sync_copy(batch_hbm.at[pl.ds(320 * (_NW - 1),
                                           _N - 320 * (_NW - 1))],
                        bt_smem.at[pl.ds(0, _N - 320 * (_NW - 1))])

    def chunk_body(c, _):
        pltpu.sync_copy(h_hbm.at[pl.ds(r0 + c * 16, 16)], rowsv)

        def row_body(e, _):
            g = bt_smem[pl.ds(c * 16 + e, 16)][0]
            for j in range(_H // 16):
                a = accv[g, pl.ds(j * 16, 16)]
                m = rowsv[e, pl.ds(j * 16, 16)]
                accv[g, pl.ds(j * 16, 16)] = jnp.maximum(a, m)
            return 0

        lax.fori_loop(0, 16, row_body, 0)
        return 0

    lax.fori_loop(0, nr // 16, chunk_body, 0)
    pltpu.sync_copy(accv, part_hbm.at[w])


_pool = pl.kernel(
    _pool_body,
    out_type=jax.ShapeDtypeStruct((_NW, _G, _H), jnp.float32),
    mesh=_mesh,
    scratch_types=[pltpu.VMEM((_G, _H), jnp.float32),
                   pltpu.VMEM((16, _H), jnp.float32),
                   pltpu.VMEM((336,), jnp.int32),
                   pltpu.SemaphoreType.DMA],
)


def _mm_kernel(agg_ref, ea_ref, wt_ref, wb_ref, s_ref, t_ref, out_ref):
    acc = jnp.dot(agg_ref[...], wt_ref[...], preferred_element_type=jnp.float32)
    acc += jnp.dot(ea_ref[...], wb_ref[...], preferred_element_type=jnp.float32)
    out_ref[...] = jnp.maximum(acc * s_ref[...] + t_ref[...], 0.0)


def _layer_update(agg, ea, wt, wb, s, t):
    hin = wt.shape[0]
    h = wt.shape[1]
    bn = 400
    return pl.pallas_call(
        _mm_kernel,
        grid=(_N // bn,),
        in_specs=[
            pl.BlockSpec((bn, hin), lambda i: (i, 0)),
            pl.BlockSpec((bn, 16), lambda i: (i, 0)),
            pl.BlockSpec((hin, h), lambda i: (0, 0)),
            pl.BlockSpec((16, h), lambda i: (0, 0)),
            pl.BlockSpec((1, h), lambda i: (0, 0)),
            pl.BlockSpec((1, h), lambda i: (0, 0)),
        ],
        out_specs=pl.BlockSpec((bn, h), lambda i: (i, 0)),
        out_shape=jax.ShapeDtypeStruct((_N, h), jnp.float32),
    )(agg, ea, wt, wb, s.reshape(1, -1), t.reshape(1, -1))


def _final_kernel(part_ref, w_ref, b_ref, out_ref):
    p = jnp.max(part_ref[...], axis=0)
    p = jnp.where(jnp.isneginf(p), 0.0, p)
    out_ref[...] = jnp.dot(p, w_ref[...],
                           preferred_element_type=jnp.float32) + b_ref[...]


def _final(part, lin_w_pad, lin_b):
    return pl.pallas_call(
        _final_kernel,
        out_shape=jax.ShapeDtypeStruct((_G, 128), jnp.float32),
    )(part, lin_w_pad, lin_b.reshape(1, 1) * jnp.ones((1, 128), jnp.float32))


def kernel(x, edge_index, edge_attr, batch, W0, b0, Wh, bh, bn_gamma, bn_beta,
           bn_mean, bn_var, lin_w, lin_b):
    src = edge_index[0]
    dst = edge_index[1]

    perm = jnp.argsort(dst).astype(jnp.int32)
    sdst = dst[perm]
    ssrc = src[perm]
    dl = sdst % _RB
    pad = jnp.zeros((_SUP + 64,), jnp.int32)
    pk_src = jnp.concatenate([ssrc * 256 + dl, pad])
    pk_dl = jnp.concatenate([dl, pad])
    edges = jnp.arange(0, _NB * _RB + 1, _RB, dtype=jnp.int32)
    starts = jnp.zeros((80,), jnp.int32).at[:_NB + 1].set(
        jnp.searchsorted(sdst, edges).astype(jnp.int32))

    ea_sorted = jnp.pad(edge_attr[perm], ((0, _C), (0, 10)))
    ea_agg = _make_agg(16, indirect=False)(ea_sorted, pk_dl, starts)

    inv_sd = bn_gamma / jnp.sqrt(bn_var + 1e-5)

    h = x
    for i in range(_L):
        hin = _D if i == 0 else _H
        agg = _make_agg(hin)(h, pk_src, starts)
        w = W0 if i == 0 else Wh[i - 1]
        b = b0 if i == 0 else bh[i - 1]
        wt = w[:hin]
        wb = jnp.pad(w[hin:], ((0, 10), (0, 0)))
        s = inv_sd[i]
        t = (b - bn_mean[i]) * s + bn_beta[i]
        h = _layer_update(agg, ea_agg, wt, wb, s, t)

    part = _pool(h, batch)
    out = _final(part, jnp.pad(lin_w, ((0, 0), (0, 127))), lin_b)
    return out[:, :1]
